# baseline (speedup 1.0000x reference)
import functools

import jax
import jax.numpy as jnp
from jax import lax
from jax.experimental import pallas as pl
from jax.experimental.pallas import tpu as pltpu
from jax.experimental.pallas import tpu_sc as plsc

E = 8
K = 2
D = 768
FF = 1024
T = 2048
NA = T * K

BT = 128
NB = NA // BT + E
NP = NB * BT

NC, NS, L = 2, 16, 16
NW = NC * NS
VPT = NA // L // NW
TPT = T // NW
SPT = NP // NW
GCH = 32

_BR = 512


def _router_body(x_ref, rw_ref, idx_ref, w_ref):
    xb = x_ref[...]
    logits = lax.dot_general(xb, rw_ref[...], (((1,), (1,)), ((), ())),
                             preferred_element_type=jnp.float32)
    idx = lax.broadcasted_iota(jnp.int32, logits.shape, 1)
    l1 = jnp.max(logits, axis=-1, keepdims=True)
    i1 = jnp.min(jnp.where(logits == l1, idx, E), axis=-1, keepdims=True)
    m1 = idx == i1
    masked = jnp.where(m1, -jnp.inf, logits)
    l2 = jnp.max(masked, axis=-1, keepdims=True)
    i2 = jnp.min(jnp.where(masked == l2, idx, E), axis=-1, keepdims=True)
    w1 = jax.nn.sigmoid(l1 - l2)
    idx_ref[...] = jnp.concatenate([i1, i2], axis=1)
    w_ref[...] = jnp.concatenate([w1, 1.0 - w1], axis=1)


CPS = NA // NS
TPS = T // NS
VPS = CPS // L
HVPS = VPS // 2


def _dispatch_body(topk_hbm, x_hbm, pos0_hbm, pos1_hbm, xs_hbm, meta_hbm,
                   topk_v, hist_sh, hist_v, myhist_v, pos0_loc, pos1_loc,
                   meta_v, xrows_v, dma_sem, row_sem):
    cid = lax.axis_index("c")
    sid = lax.axis_index("s")
    wid = sid * NC + cid
    lane = lax.iota(jnp.int32, L)

    tok0 = sid * TPS + cid * TPT
    xcp = pltpu.async_copy(x_hbm.at[pl.ds(pl.multiple_of(tok0, TPT), TPT)],
                           xrows_v, row_sem)
    pltpu.sync_copy(topk_hbm.at[pl.ds(sid * CPS, CPS)], topk_v)

    myh = jnp.zeros((L,), jnp.int32)
    h1 = jnp.zeros((L,), jnp.int32)
    dvs = []
    for v in range(VPS):
        dv = topk_v[pl.ds(v * L, L)]
        dvs.append(dv)
        for e in range(E):
            c = plsc.all_reduce_population_count(dv == e)
            myh = myh + jnp.where(lane == e, c, 0)
            if v < HVPS:
                h1 = h1 + jnp.where(lane == e, c, 0)
    myhist_v[...] = myh
    pltpu.sync_copy(myhist_v,
                    hist_sh.at[pl.ds(pl.multiple_of(sid * L, L), L)])
    plsc.subcore_barrier()
    pltpu.sync_copy(hist_sh, hist_v)

    cnt_v = jnp.zeros((L,), jnp.int32)
    pre_v = jnp.zeros((L,), jnp.int32)
    for w in range(NS):
        hw = hist_v[pl.ds(w * L, L)]
        cnt_v = cnt_v + hw
        pre_v = pre_v + jnp.where(jnp.int32(w) < sid, hw, 0)
    pre_v = pre_v + jnp.where(cid == 1, h1, 0)
    cnts = [cnt_v[e] for e in range(E)]
    nbs = [(cnts[e] + (BT - 1)) // BT for e in range(E)]
    offs, acc = [], jnp.int32(0)
    for e in range(E):
        offs.append(acc)
        acc = acc + nbs[e] * BT
    nb_tot = acc // BT
    offs_v = jnp.zeros((L,), jnp.int32)
    for e in range(E):
        offs_v = offs_v + jnp.where(lane == e, offs[e], 0)
    run_v = offs_v + pre_v
    runs = [run_v[e] for e in range(E)]

    for j in range(HVPS):
        dv0 = dvs[j]
        dv1 = dvs[j + HVPS]
        dv = jnp.where(cid == 0, dv0, dv1)
        av = (sid * CPS + cid * CPS // 2 + j * L) + lane
        slot = jnp.zeros((L,), jnp.int32)
        for e in range(E):
            m = dv == e
            pc = plsc.cumsum(m.astype(jnp.int32))
            slot = jnp.where(m, runs[e] + pc - 1, slot)
            runs[e] = runs[e] + pc[L - 1]
        tok = av // 2
        tloc = tok - tok0
        evn = (av & 1) == 0
        plsc.store_scatter(pos0_loc, [tloc], slot, mask=evn)
        plsc.store_scatter(pos1_loc, [tloc], slot,
                           mask=jnp.logical_not(evn))

    pltpu.sync_copy(pos0_loc, pos0_hbm.at[pl.ds(pl.multiple_of(tok0, TPT),
                                                TPT)])
    pltpu.sync_copy(pos1_loc, pos1_hbm.at[pl.ds(pl.multiple_of(tok0, TPT),
                                                TPT)])

    xcp.wait()
    pltpu.async_copy(xrows_v, xs_hbm.at[pos0_loc], dma_sem).wait()
    pltpu.async_copy(xrows_v, xs_hbm.at[pos1_loc], dma_sem).wait()

    @pl.when(wid == 0)
    def _():
        cum, a = [], jnp.int32(0)
        for e in range(E):
            a = a + nbs[e]
            cum.append(a)
        last_e = jnp.int32(0)
        for e in range(E):
            last_e = jnp.where(cnts[e] > 0, jnp.int32(e), last_e)
        for j in range(4):
            bv = lane + j * L
            ex = jnp.zeros((L,), jnp.int32)
            for e in range(E):
                ex = ex + (bv >= cum[e]).astype(jnp.int32)
            ex = jnp.minimum(ex, last_e)
            ex = jnp.where(bv == NB, nb_tot, ex)
            meta_v[pl.ds(j * L, L)] = ex
        pltpu.sync_copy(meta_v, meta_hbm)


def _ffn_body(meta_ref, xs_ref, gu_ref, dp_ref, ys_ref):
    b = pl.program_id(0)

    @pl.when(b < meta_ref[NB])
    def _():
        xb = xs_ref[...].astype(jnp.bfloat16)
        gu = lax.dot_general(xb, gu_ref[0].astype(jnp.bfloat16),
                             (((1,), (1,)), ((), ())),
                             preferred_element_type=jnp.float32)
        gate = gu[:, :FF]
        up = gu[:, FF:]
        h = (gate * jax.nn.sigmoid(gate) * up).astype(jnp.bfloat16)
        ys_ref[...] = lax.dot_general(h, dp_ref[0].astype(jnp.bfloat16),
                                      (((1,), (1,)), ((), ())),
                                      preferred_element_type=jnp.float32)


def _combine_sc_body(ys_hbm, pos0_hbm, pos1_hbm, yg0_hbm, yg1_hbm,
                     idx_v, buf, dma_sem):
    cid = lax.axis_index("c")
    sid = lax.axis_index("s")
    wid = sid * NC + cid
    t0 = wid * TPT
    pltpu.sync_copy(pos0_hbm.at[pl.ds(t0, TPT)], idx_v)
    pltpu.async_copy(ys_hbm.at[idx_v], buf, dma_sem).wait()
    pltpu.sync_copy(buf, yg0_hbm.at[pl.ds(t0, TPT)])
    pltpu.sync_copy(pos1_hbm.at[pl.ds(t0, TPT)], idx_v)
    pltpu.async_copy(ys_hbm.at[idx_v], buf, dma_sem).wait()
    pltpu.sync_copy(buf, yg1_hbm.at[pl.ds(t0, TPT)])


def _final_body(w_ref, yg0_ref, yg1_ref, out_ref):
    w = w_ref[...]
    out_ref[...] = (w[:, :1] * yg0_ref[...] + w[:, 1:2] * yg1_ref[...])


@functools.partial(jax.jit, static_argnames=("interpret",))
def kernel(x, router_weight, gate_up_proj, down_proj, interpret=False):
    Bb, Ss, Dd = x.shape
    xf = x.reshape(-1, Dd)

    topk, w01 = pl.pallas_call(
        _router_body,
        grid=(T // _BR,),
        in_specs=[
            pl.BlockSpec((_BR, D), lambda i: (i, 0)),
            pl.BlockSpec((E, D), lambda i: (0, 0)),
        ],
        out_specs=[
            pl.BlockSpec((_BR, K), lambda i: (i, 0)),
            pl.BlockSpec((_BR, K), lambda i: (i, 0)),
        ],
        out_shape=[
            jax.ShapeDtypeStruct((T, K), jnp.int32),
            jax.ShapeDtypeStruct((T, K), jnp.float32),
        ],
        interpret=interpret,
    )(xf, router_weight)

    mesh = plsc.VectorSubcoreMesh(core_axis_name="c", subcore_axis_name="s",
                                  num_cores=NC, num_subcores=NS)
    sc_params = pltpu.CompilerParams(needs_layout_passes=False)
    dispatch = pl.kernel(
        _dispatch_body,
        compiler_params=sc_params,
        out_type=[
            jax.ShapeDtypeStruct((T,), jnp.int32),
            jax.ShapeDtypeStruct((T,), jnp.int32),
            jax.ShapeDtypeStruct((NP, D), jnp.float32),
            jax.ShapeDtypeStruct((64,), jnp.int32),
        ],
        mesh=mesh,
        scratch_types=[
            pltpu.VMEM((CPS,), jnp.int32),
            pltpu.VMEM_SHARED((NS * L,), jnp.int32),
            pltpu.VMEM((NS * L,), jnp.int32),
            pltpu.VMEM((L,), jnp.int32),
            pltpu.VMEM((TPT,), jnp.int32),
            pltpu.VMEM((TPT,), jnp.int32),
            pltpu.VMEM((64,), jnp.int32),
            pltpu.VMEM((TPT, D), jnp.float32),
            pltpu.SemaphoreType.DMA,
            pltpu.SemaphoreType.DMA,
        ],
        interpret=interpret,
    )
    pos0, pos1, xs, meta = dispatch(topk.reshape(NA), xf)

    ys = pl.pallas_call(
        _ffn_body,
        grid_spec=pltpu.PrefetchScalarGridSpec(
            num_scalar_prefetch=1,
            grid=(NB,),
            in_specs=[
                pl.BlockSpec((BT, D), lambda b, m: (b, 0)),
                pl.BlockSpec((1, 2 * FF, D), lambda b, m: (m[b], 0, 0)),
                pl.BlockSpec((1, D, FF), lambda b, m: (m[b], 0, 0)),
            ],
            out_specs=pl.BlockSpec((BT, D), lambda b, m: (b, 0)),
        ),
        out_shape=jax.ShapeDtypeStruct((NP, D), jnp.float32),
        interpret=interpret,
    )(meta, xs, gate_up_proj, down_proj)

    combine = pl.kernel(
        _combine_sc_body,
        compiler_params=sc_params,
        out_type=[
            jax.ShapeDtypeStruct((T, D), jnp.float32),
            jax.ShapeDtypeStruct((T, D), jnp.float32),
        ],
        mesh=mesh,
        scratch_types=[
            pltpu.VMEM((TPT,), jnp.int32),
            pltpu.VMEM((TPT, D), jnp.float32),
            pltpu.SemaphoreType.DMA,
        ],
        interpret=interpret,
    )
    yg0, yg1 = combine(ys, pos0, pos1)

    out = pl.pallas_call(
        _final_body,
        grid=(T // _BR,),
        in_specs=[
            pl.BlockSpec((_BR, K), lambda i: (i, 0)),
            pl.BlockSpec((_BR, D), lambda i: (i, 0)),
            pl.BlockSpec((_BR, D), lambda i: (i, 0)),
        ],
        out_specs=pl.BlockSpec((_BR, D), lambda i: (i, 0)),
        out_shape=jax.ShapeDtypeStruct((T, D), jnp.float32),
        interpret=interpret,
    )(w01, yg0, yg1)

    return out.reshape(Bb, Ss, Dd)


# speedup vs baseline: 1.0096x; 1.0096x over previous
import functools

import jax
import jax.numpy as jnp
from jax import lax
from jax.experimental import pallas as pl
from jax.experimental.pallas import tpu as pltpu
from jax.experimental.pallas import tpu_sc as plsc

E = 8
K = 2
D = 768
FF = 1024
T = 2048
NA = T * K

BT = 128
NB = NA // BT + E
NP = NB * BT

NC, NS, L = 2, 16, 16
NW = NC * NS
VPT = NA // L // NW
TPT = T // NW
SPT = NP // NW
GCH = 32

_BR = 2048


def _router_body(x_ref, rw_ref, idx_ref, w_ref):
    xb = x_ref[...]
    logits = lax.dot_general(xb, rw_ref[...], (((1,), (1,)), ((), ())),
                             preferred_element_type=jnp.float32)
    idx = lax.broadcasted_iota(jnp.int32, logits.shape, 1)
    l1 = jnp.max(logits, axis=-1, keepdims=True)
    i1 = jnp.min(jnp.where(logits == l1, idx, E), axis=-1, keepdims=True)
    m1 = idx == i1
    masked = jnp.where(m1, -jnp.inf, logits)
    l2 = jnp.max(masked, axis=-1, keepdims=True)
    i2 = jnp.min(jnp.where(masked == l2, idx, E), axis=-1, keepdims=True)
    w1 = jax.nn.sigmoid(l1 - l2)
    idx_ref[...] = jnp.concatenate([i1, i2], axis=1)
    w_ref[...] = jnp.concatenate([w1, 1.0 - w1], axis=1)


CPS = NA // NS
TPS = T // NS
VPS = CPS // L
HVPS = VPS // 2


def _dispatch_body(topk_hbm, x_hbm, pos0_hbm, pos1_hbm, xs_hbm, meta_hbm,
                   topk_v, hist_sh, hist_v, myhist_v, pos0_loc, pos1_loc,
                   meta_v, xrows_v, dma_sem, row_sem):
    cid = lax.axis_index("c")
    sid = lax.axis_index("s")
    wid = sid * NC + cid
    lane = lax.iota(jnp.int32, L)

    tok0 = sid * TPS + cid * TPT
    xcp = pltpu.async_copy(x_hbm.at[pl.ds(pl.multiple_of(tok0, TPT), TPT)],
                           xrows_v, row_sem)
    pltpu.sync_copy(topk_hbm.at[pl.ds(sid * CPS, CPS)], topk_v)

    myh = jnp.zeros((L,), jnp.int32)
    h1 = jnp.zeros((L,), jnp.int32)
    dvs = []
    for v in range(VPS):
        dv = topk_v[pl.ds(v * L, L)]
        dvs.append(dv)
        for e in range(E):
            c = plsc.all_reduce_population_count(dv == e)
            myh = myh + jnp.where(lane == e, c, 0)
            if v < HVPS:
                h1 = h1 + jnp.where(lane == e, c, 0)
    myhist_v[...] = myh
    pltpu.sync_copy(myhist_v,
                    hist_sh.at[pl.ds(pl.multiple_of(sid * L, L), L)])
    plsc.subcore_barrier()
    pltpu.sync_copy(hist_sh, hist_v)

    cnt_v = jnp.zeros((L,), jnp.int32)
    pre_v = jnp.zeros((L,), jnp.int32)
    for w in range(NS):
        hw = hist_v[pl.ds(w * L, L)]
        cnt_v = cnt_v + hw
        pre_v = pre_v + jnp.where(jnp.int32(w) < sid, hw, 0)
    pre_v = pre_v + jnp.where(cid == 1, h1, 0)
    cnts = [cnt_v[e] for e in range(E)]
    nbs = [(cnts[e] + (BT - 1)) // BT for e in range(E)]
    offs, acc = [], jnp.int32(0)
    for e in range(E):
        offs.append(acc)
        acc = acc + nbs[e] * BT
    nb_tot = acc // BT
    offs_v = jnp.zeros((L,), jnp.int32)
    for e in range(E):
        offs_v = offs_v + jnp.where(lane == e, offs[e], 0)
    run_v = offs_v + pre_v
    runs = [run_v[e] for e in range(E)]

    for j in range(HVPS):
        dv0 = dvs[j]
        dv1 = dvs[j + HVPS]
        dv = jnp.where(cid == 0, dv0, dv1)
        av = (sid * CPS + cid * CPS // 2 + j * L) + lane
        slot = jnp.zeros((L,), jnp.int32)
        for e in range(E):
            m = dv == e
            pc = plsc.cumsum(m.astype(jnp.int32))
            slot = jnp.where(m, runs[e] + pc - 1, slot)
            runs[e] = runs[e] + pc[L - 1]
        tok = av // 2
        tloc = tok - tok0
        evn = (av & 1) == 0
        plsc.store_scatter(pos0_loc, [tloc], slot, mask=evn)
        plsc.store_scatter(pos1_loc, [tloc], slot,
                           mask=jnp.logical_not(evn))

    pltpu.sync_copy(pos0_loc, pos0_hbm.at[pl.ds(pl.multiple_of(tok0, TPT),
                                                TPT)])
    pltpu.sync_copy(pos1_loc, pos1_hbm.at[pl.ds(pl.multiple_of(tok0, TPT),
                                                TPT)])

    xcp.wait()
    pltpu.async_copy(xrows_v, xs_hbm.at[pos0_loc], dma_sem).wait()
    pltpu.async_copy(xrows_v, xs_hbm.at[pos1_loc], dma_sem).wait()

    @pl.when(wid == 0)
    def _():
        cum, a = [], jnp.int32(0)
        for e in range(E):
            a = a + nbs[e]
            cum.append(a)
        last_e = jnp.int32(0)
        for e in range(E):
            last_e = jnp.where(cnts[e] > 0, jnp.int32(e), last_e)
        for j in range(4):
            bv = lane + j * L
            ex = jnp.zeros((L,), jnp.int32)
            for e in range(E):
                ex = ex + (bv >= cum[e]).astype(jnp.int32)
            ex = jnp.minimum(ex, last_e)
            ex = jnp.where(bv == NB, nb_tot, ex)
            meta_v[pl.ds(j * L, L)] = ex
        pltpu.sync_copy(meta_v, meta_hbm)


def _ffn_body(meta_ref, xs_ref, gu_ref, dp_ref, ys_ref):
    b = pl.program_id(0)

    @pl.when(b < meta_ref[NB])
    def _():
        xb = xs_ref[...]
        gu = lax.dot_general(xb, gu_ref[0], (((1,), (1,)), ((), ())),
                             preferred_element_type=jnp.float32)
        gate = gu[:, :FF]
        up = gu[:, FF:]
        h = gate * jax.nn.sigmoid(gate) * up
        ys_ref[...] = lax.dot_general(h, dp_ref[0], (((1,), (1,)), ((), ())),
                                      preferred_element_type=jnp.float32)


def _combine_sc_body(ys_hbm, pos0_hbm, pos1_hbm, yg0_hbm, yg1_hbm,
                     idx_v, buf, dma_sem):
    cid = lax.axis_index("c")
    sid = lax.axis_index("s")
    wid = sid * NC + cid
    t0 = wid * TPT
    pltpu.sync_copy(pos0_hbm.at[pl.ds(t0, TPT)], idx_v)
    pltpu.async_copy(ys_hbm.at[idx_v], buf, dma_sem).wait()
    pltpu.sync_copy(buf, yg0_hbm.at[pl.ds(t0, TPT)])
    pltpu.sync_copy(pos1_hbm.at[pl.ds(t0, TPT)], idx_v)
    pltpu.async_copy(ys_hbm.at[idx_v], buf, dma_sem).wait()
    pltpu.sync_copy(buf, yg1_hbm.at[pl.ds(t0, TPT)])


def _final_body(w_ref, yg0_ref, yg1_ref, out_ref):
    w = w_ref[...]
    out_ref[...] = (w[:, :1] * yg0_ref[...] + w[:, 1:2] * yg1_ref[...])


@functools.partial(jax.jit, static_argnames=("interpret",))
def kernel(x, router_weight, gate_up_proj, down_proj, interpret=False):
    Bb, Ss, Dd = x.shape
    xf = x.reshape(-1, Dd)

    topk, w01 = pl.pallas_call(
        _router_body,
        grid=(T // _BR,),
        in_specs=[
            pl.BlockSpec((_BR, D), lambda i: (i, 0)),
            pl.BlockSpec((E, D), lambda i: (0, 0)),
        ],
        out_specs=[
            pl.BlockSpec((_BR, K), lambda i: (i, 0)),
            pl.BlockSpec((_BR, K), lambda i: (i, 0)),
        ],
        out_shape=[
            jax.ShapeDtypeStruct((T, K), jnp.int32),
            jax.ShapeDtypeStruct((T, K), jnp.float32),
        ],
        interpret=interpret,
    )(xf, router_weight)

    mesh = plsc.VectorSubcoreMesh(core_axis_name="c", subcore_axis_name="s",
                                  num_cores=NC, num_subcores=NS)
    sc_params = pltpu.CompilerParams(needs_layout_passes=False)
    dispatch = pl.kernel(
        _dispatch_body,
        compiler_params=sc_params,
        out_type=[
            jax.ShapeDtypeStruct((T,), jnp.int32),
            jax.ShapeDtypeStruct((T,), jnp.int32),
            jax.ShapeDtypeStruct((NP, D), jnp.float32),
            jax.ShapeDtypeStruct((64,), jnp.int32),
        ],
        mesh=mesh,
        scratch_types=[
            pltpu.VMEM((CPS,), jnp.int32),
            pltpu.VMEM_SHARED((NS * L,), jnp.int32),
            pltpu.VMEM((NS * L,), jnp.int32),
            pltpu.VMEM((L,), jnp.int32),
            pltpu.VMEM((TPT,), jnp.int32),
            pltpu.VMEM((TPT,), jnp.int32),
            pltpu.VMEM((64,), jnp.int32),
            pltpu.VMEM((TPT, D), jnp.float32),
            pltpu.SemaphoreType.DMA,
            pltpu.SemaphoreType.DMA,
        ],
        interpret=interpret,
    )
    pos0, pos1, xs, meta = dispatch(topk.reshape(NA), xf)

    ys = pl.pallas_call(
        _ffn_body,
        grid_spec=pltpu.PrefetchScalarGridSpec(
            num_scalar_prefetch=1,
            grid=(NB,),
            in_specs=[
                pl.BlockSpec((BT, D), lambda b, m: (b, 0)),
                pl.BlockSpec((1, 2 * FF, D), lambda b, m: (m[b], 0, 0)),
                pl.BlockSpec((1, D, FF), lambda b, m: (m[b], 0, 0)),
            ],
            out_specs=pl.BlockSpec((BT, D), lambda b, m: (b, 0)),
        ),
        out_shape=jax.ShapeDtypeStruct((NP, D), jnp.float32),
        interpret=interpret,
    )(meta, xs, gate_up_proj, down_proj)

    combine = pl.kernel(
        _combine_sc_body,
        compiler_params=sc_params,
        out_type=[
            jax.ShapeDtypeStruct((T, D), jnp.float32),
            jax.ShapeDtypeStruct((T, D), jnp.float32),
        ],
        mesh=mesh,
        scratch_types=[
            pltpu.VMEM((TPT,), jnp.int32),
            pltpu.VMEM((TPT, D), jnp.float32),
            pltpu.SemaphoreType.DMA,
        ],
        interpret=interpret,
    )
    yg0, yg1 = combine(ys, pos0, pos1)

    out = pl.pallas_call(
        _final_body,
        grid=(T // _BR,),
        in_specs=[
            pl.BlockSpec((_BR, K), lambda i: (i, 0)),
            pl.BlockSpec((_BR, D), lambda i: (i, 0)),
            pl.BlockSpec((_BR, D), lambda i: (i, 0)),
        ],
        out_specs=pl.BlockSpec((_BR, D), lambda i: (i, 0)),
        out_shape=jax.ShapeDtypeStruct((T, D), jnp.float32),
        interpret=interpret,
    )(w01, yg0, yg1)

    return out.reshape(Bb, Ss, Dd)


# speedup vs baseline: 1.2486x; 1.2367x over previous
import functools

import jax
import jax.numpy as jnp
from jax import lax
from jax.experimental import pallas as pl
from jax.experimental.pallas import tpu as pltpu
from jax.experimental.pallas import tpu_sc as plsc

E = 8
K = 2
D = 768
FF = 1024
T = 2048
NA = T * K

BT = 256
NB = NA // BT + E
NP = NB * BT

NC, NS, L = 2, 16, 16
NW = NC * NS
VPT = NA // L // NW
TPT = T // NW
SPT = NP // NW
GCH = 32

_BR = 2048


def _router_body(x_ref, rw_ref, idx_ref, w_ref):
    xb = x_ref[...]
    logits = lax.dot_general(xb, rw_ref[...], (((1,), (1,)), ((), ())),
                             preferred_element_type=jnp.float32)
    idx = lax.broadcasted_iota(jnp.int32, logits.shape, 1)
    l1 = jnp.max(logits, axis=-1, keepdims=True)
    i1 = jnp.min(jnp.where(logits == l1, idx, E), axis=-1, keepdims=True)
    m1 = idx == i1
    masked = jnp.where(m1, -jnp.inf, logits)
    l2 = jnp.max(masked, axis=-1, keepdims=True)
    i2 = jnp.min(jnp.where(masked == l2, idx, E), axis=-1, keepdims=True)
    w1 = jax.nn.sigmoid(l1 - l2)
    idx_ref[...] = jnp.concatenate([i1, i2], axis=1)
    w_ref[...] = jnp.concatenate([w1, 1.0 - w1], axis=1)


CPS = NA // NS
TPS = T // NS
VPS = CPS // L
HVPS = VPS // 2


def _dispatch_body(topk_hbm, x_hbm, pos0_hbm, pos1_hbm, xs_hbm, meta_hbm,
                   topk_v, hist_sh, hist_v, myhist_v, pos0_loc, pos1_loc,
                   meta_v, xrows_v, dma_sem, row_sem):
    cid = lax.axis_index("c")
    sid = lax.axis_index("s")
    wid = sid * NC + cid
    lane = lax.iota(jnp.int32, L)

    tok0 = sid * TPS + cid * TPT
    xcp = pltpu.async_copy(x_hbm.at[pl.ds(pl.multiple_of(tok0, TPT), TPT)],
                           xrows_v, row_sem)
    pltpu.sync_copy(topk_hbm.at[pl.ds(sid * CPS, CPS)], topk_v)

    myh = jnp.zeros((L,), jnp.int32)
    h1 = jnp.zeros((L,), jnp.int32)
    dvs = []
    for v in range(VPS):
        dv = topk_v[pl.ds(v * L, L)]
        dvs.append(dv)
        for e in range(E):
            c = plsc.all_reduce_population_count(dv == e)
            myh = myh + jnp.where(lane == e, c, 0)
            if v < HVPS:
                h1 = h1 + jnp.where(lane == e, c, 0)
    myhist_v[...] = myh
    pltpu.sync_copy(myhist_v,
                    hist_sh.at[pl.ds(pl.multiple_of(sid * L, L), L)])
    plsc.subcore_barrier()
    pltpu.sync_copy(hist_sh, hist_v)

    cnt_v = jnp.zeros((L,), jnp.int32)
    pre_v = jnp.zeros((L,), jnp.int32)
    for w in range(NS):
        hw = hist_v[pl.ds(w * L, L)]
        cnt_v = cnt_v + hw
        pre_v = pre_v + jnp.where(jnp.int32(w) < sid, hw, 0)
    pre_v = pre_v + jnp.where(cid == 1, h1, 0)
    cnts = [cnt_v[e] for e in range(E)]
    nbs = [(cnts[e] + (BT - 1)) // BT for e in range(E)]
    offs, acc = [], jnp.int32(0)
    for e in range(E):
        offs.append(acc)
        acc = acc + nbs[e] * BT
    nb_tot = acc // BT
    offs_v = jnp.zeros((L,), jnp.int32)
    for e in range(E):
        offs_v = offs_v + jnp.where(lane == e, offs[e], 0)
    run_v = offs_v + pre_v
    runs = [run_v[e] for e in range(E)]

    for j in range(HVPS):
        dv0 = dvs[j]
        dv1 = dvs[j + HVPS]
        dv = jnp.where(cid == 0, dv0, dv1)
        av = (sid * CPS + cid * CPS // 2 + j * L) + lane
        slot = jnp.zeros((L,), jnp.int32)
        for e in range(E):
            m = dv == e
            pc = plsc.cumsum(m.astype(jnp.int32))
            slot = jnp.where(m, runs[e] + pc - 1, slot)
            runs[e] = runs[e] + pc[L - 1]
        tok = av // 2
        tloc = tok - tok0
        evn = (av & 1) == 0
        plsc.store_scatter(pos0_loc, [tloc], slot, mask=evn)
        plsc.store_scatter(pos1_loc, [tloc], slot,
                           mask=jnp.logical_not(evn))

    pltpu.sync_copy(pos0_loc, pos0_hbm.at[pl.ds(pl.multiple_of(tok0, TPT),
                                                TPT)])
    pltpu.sync_copy(pos1_loc, pos1_hbm.at[pl.ds(pl.multiple_of(tok0, TPT),
                                                TPT)])

    xcp.wait()
    pltpu.async_copy(xrows_v, xs_hbm.at[pos0_loc], dma_sem).wait()
    pltpu.async_copy(xrows_v, xs_hbm.at[pos1_loc], dma_sem).wait()

    @pl.when(wid == 0)
    def _():
        cum, a = [], jnp.int32(0)
        for e in range(E):
            a = a + nbs[e]
            cum.append(a)
        last_e = jnp.int32(0)
        for e in range(E):
            last_e = jnp.where(cnts[e] > 0, jnp.int32(e), last_e)
        for j in range(4):
            bv = lane + j * L
            ex = jnp.zeros((L,), jnp.int32)
            for e in range(E):
                ex = ex + (bv >= cum[e]).astype(jnp.int32)
            ex = jnp.minimum(ex, last_e)
            ex = jnp.where(bv == NB, nb_tot, ex)
            meta_v[pl.ds(j * L, L)] = ex
        pltpu.sync_copy(meta_v, meta_hbm)


def _ffn_body(meta_ref, xs_ref, gu_ref, dp_ref, ys_ref):
    b = pl.program_id(0)

    @pl.when(b < meta_ref[NB])
    def _():
        xb = xs_ref[...]
        gu = lax.dot_general(xb, gu_ref[0], (((1,), (1,)), ((), ())),
                             preferred_element_type=jnp.float32)
        gate = gu[:, :FF]
        up = gu[:, FF:]
        h = gate * jax.nn.sigmoid(gate) * up
        ys_ref[...] = lax.dot_general(h, dp_ref[0], (((1,), (1,)), ((), ())),
                                      preferred_element_type=jnp.float32)


def _combine_sc_body(ys_hbm, pos0_hbm, pos1_hbm, yg0_hbm, yg1_hbm,
                     idx_v, buf, dma_sem):
    cid = lax.axis_index("c")
    sid = lax.axis_index("s")
    wid = sid * NC + cid
    t0 = wid * TPT
    pltpu.sync_copy(pos0_hbm.at[pl.ds(t0, TPT)], idx_v)
    pltpu.async_copy(ys_hbm.at[idx_v], buf, dma_sem).wait()
    pltpu.sync_copy(buf, yg0_hbm.at[pl.ds(t0, TPT)])
    pltpu.sync_copy(pos1_hbm.at[pl.ds(t0, TPT)], idx_v)
    pltpu.async_copy(ys_hbm.at[idx_v], buf, dma_sem).wait()
    pltpu.sync_copy(buf, yg1_hbm.at[pl.ds(t0, TPT)])


def _final_body(w_ref, yg0_ref, yg1_ref, out_ref):
    w = w_ref[...]
    out_ref[...] = (w[:, :1] * yg0_ref[...] + w[:, 1:2] * yg1_ref[...])


@functools.partial(jax.jit, static_argnames=("interpret",))
def kernel(x, router_weight, gate_up_proj, down_proj, interpret=False):
    Bb, Ss, Dd = x.shape
    xf = x.reshape(-1, Dd)

    topk, w01 = pl.pallas_call(
        _router_body,
        grid=(T // _BR,),
        in_specs=[
            pl.BlockSpec((_BR, D), lambda i: (i, 0)),
            pl.BlockSpec((E, D), lambda i: (0, 0)),
        ],
        out_specs=[
            pl.BlockSpec((_BR, K), lambda i: (i, 0)),
            pl.BlockSpec((_BR, K), lambda i: (i, 0)),
        ],
        out_shape=[
            jax.ShapeDtypeStruct((T, K), jnp.int32),
            jax.ShapeDtypeStruct((T, K), jnp.float32),
        ],
        interpret=interpret,
    )(xf, router_weight)

    mesh = plsc.VectorSubcoreMesh(core_axis_name="c", subcore_axis_name="s",
                                  num_cores=NC, num_subcores=NS)
    sc_params = pltpu.CompilerParams(needs_layout_passes=False)
    dispatch = pl.kernel(
        _dispatch_body,
        compiler_params=sc_params,
        out_type=[
            jax.ShapeDtypeStruct((T,), jnp.int32),
            jax.ShapeDtypeStruct((T,), jnp.int32),
            jax.ShapeDtypeStruct((NP, D), jnp.float32),
            jax.ShapeDtypeStruct((64,), jnp.int32),
        ],
        mesh=mesh,
        scratch_types=[
            pltpu.VMEM((CPS,), jnp.int32),
            pltpu.VMEM_SHARED((NS * L,), jnp.int32),
            pltpu.VMEM((NS * L,), jnp.int32),
            pltpu.VMEM((L,), jnp.int32),
            pltpu.VMEM((TPT,), jnp.int32),
            pltpu.VMEM((TPT,), jnp.int32),
            pltpu.VMEM((64,), jnp.int32),
            pltpu.VMEM((TPT, D), jnp.float32),
            pltpu.SemaphoreType.DMA,
            pltpu.SemaphoreType.DMA,
        ],
        interpret=interpret,
    )
    pos0, pos1, xs, meta = dispatch(topk.reshape(NA), xf)

    ys = pl.pallas_call(
        _ffn_body,
        grid_spec=pltpu.PrefetchScalarGridSpec(
            num_scalar_prefetch=1,
            grid=(NB,),
            in_specs=[
                pl.BlockSpec((BT, D), lambda b, m: (b, 0)),
                pl.BlockSpec((1, 2 * FF, D), lambda b, m: (m[b], 0, 0)),
                pl.BlockSpec((1, D, FF), lambda b, m: (m[b], 0, 0)),
            ],
            out_specs=pl.BlockSpec((BT, D), lambda b, m: (b, 0)),
        ),
        out_shape=jax.ShapeDtypeStruct((NP, D), jnp.float32),
        interpret=interpret,
    )(meta, xs, gate_up_proj, down_proj)

    combine = pl.kernel(
        _combine_sc_body,
        compiler_params=sc_params,
        out_type=[
            jax.ShapeDtypeStruct((T, D), jnp.float32),
            jax.ShapeDtypeStruct((T, D), jnp.float32),
        ],
        mesh=mesh,
        scratch_types=[
            pltpu.VMEM((TPT,), jnp.int32),
            pltpu.VMEM((TPT, D), jnp.float32),
            pltpu.SemaphoreType.DMA,
        ],
        interpret=interpret,
    )
    yg0, yg1 = combine(ys, pos0, pos1)

    out = pl.pallas_call(
        _final_body,
        grid=(T // _BR,),
        in_specs=[
            pl.BlockSpec((_BR, K), lambda i: (i, 0)),
            pl.BlockSpec((_BR, D), lambda i: (i, 0)),
            pl.BlockSpec((_BR, D), lambda i: (i, 0)),
        ],
        out_specs=pl.BlockSpec((_BR, D), lambda i: (i, 0)),
        out_shape=jax.ShapeDtypeStruct((T, D), jnp.float32),
        interpret=interpret,
    )(w01, yg0, yg1)

    return out.reshape(Bb, Ss, Dd)


# speedup vs baseline: 1.3220x; 1.0587x over previous
import functools

import jax
import jax.numpy as jnp
from jax import lax
from jax.experimental import pallas as pl
from jax.experimental.pallas import tpu as pltpu
from jax.experimental.pallas import tpu_sc as plsc

E = 8
K = 2
D = 768
FF = 1024
T = 2048
NA = T * K

BT = 512
NB = NA // BT + E
NP = NB * BT

NC, NS, L = 2, 16, 16
NW = NC * NS
VPT = NA // L // NW
TPT = T // NW
SPT = NP // NW
GCH = 32

_BR = 2048


def _router_body(x_ref, rw_ref, idx_ref, w_ref):
    xb = x_ref[...]
    logits = lax.dot_general(xb, rw_ref[...], (((1,), (1,)), ((), ())),
                             preferred_element_type=jnp.float32)
    idx = lax.broadcasted_iota(jnp.int32, logits.shape, 1)
    l1 = jnp.max(logits, axis=-1, keepdims=True)
    i1 = jnp.min(jnp.where(logits == l1, idx, E), axis=-1, keepdims=True)
    m1 = idx == i1
    masked = jnp.where(m1, -jnp.inf, logits)
    l2 = jnp.max(masked, axis=-1, keepdims=True)
    i2 = jnp.min(jnp.where(masked == l2, idx, E), axis=-1, keepdims=True)
    w1 = jax.nn.sigmoid(l1 - l2)
    idx_ref[...] = jnp.concatenate([i1, i2], axis=1)
    w_ref[...] = jnp.concatenate([w1, 1.0 - w1], axis=1)


CPS = NA // NS
TPS = T // NS
VPS = CPS // L
HVPS = VPS // 2


def _dispatch_body(topk_hbm, x_hbm, pos0_hbm, pos1_hbm, xs_hbm, meta_hbm,
                   topk_v, hist_sh, hist_v, myhist_v, pos0_loc, pos1_loc,
                   meta_v, xrows_v, dma_sem, row_sem):
    cid = lax.axis_index("c")
    sid = lax.axis_index("s")
    wid = sid * NC + cid
    lane = lax.iota(jnp.int32, L)

    tok0 = sid * TPS + cid * TPT
    xcp = pltpu.async_copy(x_hbm.at[pl.ds(pl.multiple_of(tok0, TPT), TPT)],
                           xrows_v, row_sem)
    pltpu.sync_copy(topk_hbm.at[pl.ds(sid * CPS, CPS)], topk_v)

    myh = jnp.zeros((L,), jnp.int32)
    h1 = jnp.zeros((L,), jnp.int32)
    dvs = []
    for v in range(VPS):
        dv = topk_v[pl.ds(v * L, L)]
        dvs.append(dv)
        for e in range(E):
            c = plsc.all_reduce_population_count(dv == e)
            myh = myh + jnp.where(lane == e, c, 0)
            if v < HVPS:
                h1 = h1 + jnp.where(lane == e, c, 0)
    myhist_v[...] = myh
    pltpu.sync_copy(myhist_v,
                    hist_sh.at[pl.ds(pl.multiple_of(sid * L, L), L)])
    plsc.subcore_barrier()
    pltpu.sync_copy(hist_sh, hist_v)

    cnt_v = jnp.zeros((L,), jnp.int32)
    pre_v = jnp.zeros((L,), jnp.int32)
    for w in range(NS):
        hw = hist_v[pl.ds(w * L, L)]
        cnt_v = cnt_v + hw
        pre_v = pre_v + jnp.where(jnp.int32(w) < sid, hw, 0)
    pre_v = pre_v + jnp.where(cid == 1, h1, 0)
    cnts = [cnt_v[e] for e in range(E)]
    nbs = [(cnts[e] + (BT - 1)) // BT for e in range(E)]
    offs, acc = [], jnp.int32(0)
    for e in range(E):
        offs.append(acc)
        acc = acc + nbs[e] * BT
    nb_tot = acc // BT
    offs_v = jnp.zeros((L,), jnp.int32)
    for e in range(E):
        offs_v = offs_v + jnp.where(lane == e, offs[e], 0)
    run_v = offs_v + pre_v
    runs = [run_v[e] for e in range(E)]

    for j in range(HVPS):
        dv0 = dvs[j]
        dv1 = dvs[j + HVPS]
        dv = jnp.where(cid == 0, dv0, dv1)
        av = (sid * CPS + cid * CPS // 2 + j * L) + lane
        slot = jnp.zeros((L,), jnp.int32)
        for e in range(E):
            m = dv == e
            pc = plsc.cumsum(m.astype(jnp.int32))
            slot = jnp.where(m, runs[e] + pc - 1, slot)
            runs[e] = runs[e] + pc[L - 1]
        tok = av // 2
        tloc = tok - tok0
        evn = (av & 1) == 0
        plsc.store_scatter(pos0_loc, [tloc], slot, mask=evn)
        plsc.store_scatter(pos1_loc, [tloc], slot,
                           mask=jnp.logical_not(evn))

    pltpu.sync_copy(pos0_loc, pos0_hbm.at[pl.ds(pl.multiple_of(tok0, TPT),
                                                TPT)])
    pltpu.sync_copy(pos1_loc, pos1_hbm.at[pl.ds(pl.multiple_of(tok0, TPT),
                                                TPT)])

    xcp.wait()
    pltpu.async_copy(xrows_v, xs_hbm.at[pos0_loc], dma_sem).wait()
    pltpu.async_copy(xrows_v, xs_hbm.at[pos1_loc], dma_sem).wait()

    @pl.when(wid == 0)
    def _():
        cum, a = [], jnp.int32(0)
        for e in range(E):
            a = a + nbs[e]
            cum.append(a)
        last_e = jnp.int32(0)
        for e in range(E):
            last_e = jnp.where(cnts[e] > 0, jnp.int32(e), last_e)
        for j in range(4):
            bv = lane + j * L
            ex = jnp.zeros((L,), jnp.int32)
            for e in range(E):
                ex = ex + (bv >= cum[e]).astype(jnp.int32)
            ex = jnp.minimum(ex, last_e)
            ex = jnp.where(bv == NB, nb_tot, ex)
            meta_v[pl.ds(j * L, L)] = ex
        pltpu.sync_copy(meta_v, meta_hbm)


def _ffn_body(meta_ref, xs_ref, gu_ref, dp_ref, ys_ref):
    b = pl.program_id(0)

    @pl.when(b < meta_ref[NB])
    def _():
        xb = xs_ref[...]
        gu = lax.dot_general(xb, gu_ref[0], (((1,), (1,)), ((), ())),
                             preferred_element_type=jnp.float32)
        gate = gu[:, :FF]
        up = gu[:, FF:]
        h = gate * jax.nn.sigmoid(gate) * up
        ys_ref[...] = lax.dot_general(h, dp_ref[0], (((1,), (1,)), ((), ())),
                                      preferred_element_type=jnp.float32)


def _combine_sc_body(ys_hbm, pos0_hbm, pos1_hbm, yg0_hbm, yg1_hbm,
                     idx_v, buf, dma_sem):
    cid = lax.axis_index("c")
    sid = lax.axis_index("s")
    wid = sid * NC + cid
    t0 = wid * TPT
    pltpu.sync_copy(pos0_hbm.at[pl.ds(t0, TPT)], idx_v)
    pltpu.async_copy(ys_hbm.at[idx_v], buf, dma_sem).wait()
    pltpu.sync_copy(buf, yg0_hbm.at[pl.ds(t0, TPT)])
    pltpu.sync_copy(pos1_hbm.at[pl.ds(t0, TPT)], idx_v)
    pltpu.async_copy(ys_hbm.at[idx_v], buf, dma_sem).wait()
    pltpu.sync_copy(buf, yg1_hbm.at[pl.ds(t0, TPT)])


def _final_body(w_ref, yg0_ref, yg1_ref, out_ref):
    w = w_ref[...]
    out_ref[...] = (w[:, :1] * yg0_ref[...] + w[:, 1:2] * yg1_ref[...])


@functools.partial(jax.jit, static_argnames=("interpret",))
def kernel(x, router_weight, gate_up_proj, down_proj, interpret=False):
    Bb, Ss, Dd = x.shape
    xf = x.reshape(-1, Dd)

    topk, w01 = pl.pallas_call(
        _router_body,
        grid=(T // _BR,),
        in_specs=[
            pl.BlockSpec((_BR, D), lambda i: (i, 0)),
            pl.BlockSpec((E, D), lambda i: (0, 0)),
        ],
        out_specs=[
            pl.BlockSpec((_BR, K), lambda i: (i, 0)),
            pl.BlockSpec((_BR, K), lambda i: (i, 0)),
        ],
        out_shape=[
            jax.ShapeDtypeStruct((T, K), jnp.int32),
            jax.ShapeDtypeStruct((T, K), jnp.float32),
        ],
        interpret=interpret,
    )(xf, router_weight)

    mesh = plsc.VectorSubcoreMesh(core_axis_name="c", subcore_axis_name="s",
                                  num_cores=NC, num_subcores=NS)
    sc_params = pltpu.CompilerParams(needs_layout_passes=False)
    dispatch = pl.kernel(
        _dispatch_body,
        compiler_params=sc_params,
        out_type=[
            jax.ShapeDtypeStruct((T,), jnp.int32),
            jax.ShapeDtypeStruct((T,), jnp.int32),
            jax.ShapeDtypeStruct((NP, D), jnp.float32),
            jax.ShapeDtypeStruct((64,), jnp.int32),
        ],
        mesh=mesh,
        scratch_types=[
            pltpu.VMEM((CPS,), jnp.int32),
            pltpu.VMEM_SHARED((NS * L,), jnp.int32),
            pltpu.VMEM((NS * L,), jnp.int32),
            pltpu.VMEM((L,), jnp.int32),
            pltpu.VMEM((TPT,), jnp.int32),
            pltpu.VMEM((TPT,), jnp.int32),
            pltpu.VMEM((64,), jnp.int32),
            pltpu.VMEM((TPT, D), jnp.float32),
            pltpu.SemaphoreType.DMA,
            pltpu.SemaphoreType.DMA,
        ],
        interpret=interpret,
    )
    pos0, pos1, xs, meta = dispatch(topk.reshape(NA), xf)

    ys = pl.pallas_call(
        _ffn_body,
        grid_spec=pltpu.PrefetchScalarGridSpec(
            num_scalar_prefetch=1,
            grid=(NB,),
            in_specs=[
                pl.BlockSpec((BT, D), lambda b, m: (b, 0)),
                pl.BlockSpec((1, 2 * FF, D), lambda b, m: (m[b], 0, 0)),
                pl.BlockSpec((1, D, FF), lambda b, m: (m[b], 0, 0)),
            ],
            out_specs=pl.BlockSpec((BT, D), lambda b, m: (b, 0)),
        ),
        out_shape=jax.ShapeDtypeStruct((NP, D), jnp.float32),
        interpret=interpret,
    )(meta, xs, gate_up_proj, down_proj)

    combine = pl.kernel(
        _combine_sc_body,
        compiler_params=sc_params,
        out_type=[
            jax.ShapeDtypeStruct((T, D), jnp.float32),
            jax.ShapeDtypeStruct((T, D), jnp.float32),
        ],
        mesh=mesh,
        scratch_types=[
            pltpu.VMEM((TPT,), jnp.int32),
            pltpu.VMEM((TPT, D), jnp.float32),
            pltpu.SemaphoreType.DMA,
        ],
        interpret=interpret,
    )
    yg0, yg1 = combine(ys, pos0, pos1)

    out = pl.pallas_call(
        _final_body,
        grid=(T // _BR,),
        in_specs=[
            pl.BlockSpec((_BR, K), lambda i: (i, 0)),
            pl.BlockSpec((_BR, D), lambda i: (i, 0)),
            pl.BlockSpec((_BR, D), lambda i: (i, 0)),
        ],
        out_specs=pl.BlockSpec((_BR, D), lambda i: (i, 0)),
        out_shape=jax.ShapeDtypeStruct((T, D), jnp.float32),
        interpret=interpret,
    )(w01, yg0, yg1)

    return out.reshape(Bb, Ss, Dd)
